# baseline (speedup 1.0000x reference)
import jax
import jax.numpy as jnp
from jax.experimental import pallas as pl
from jax.experimental.pallas import tpu as pltpu

OBJ = 1024
ATTR = 8
DIM = 256
BATCH = 4
SEQ = OBJ * ATTR

_CH = 64


def _body(table_ref, out_ref, eb, sem):
    cps = []
    for j in range(OBJ // _CH):
        t = table_ref[pl.ds(j * _CH, _CH), :]
        lo = j * _CH * ATTR
        eb[pl.ds(lo, _CH * ATTR), :] = jnp.broadcast_to(
            t[:, None, :], (_CH, ATTR, DIM)
        ).reshape(_CH * ATTR, DIM)
        for b in range(BATCH):
            c = pltpu.make_async_copy(
                eb.at[pl.ds(lo, _CH * ATTR)],
                out_ref.at[b, pl.ds(lo, _CH * ATTR)],
                sem,
            )
            c.start()
            cps.append(c)
    for c in cps:
        c.wait()


def kernel(x, E_object_index):
    del x
    return pl.pallas_call(
        _body,
        in_specs=[pl.BlockSpec((OBJ, DIM), lambda: (0, 0))],
        out_specs=pl.BlockSpec(memory_space=pl.ANY),
        out_shape=jax.ShapeDtypeStruct((BATCH, SEQ, DIM), jnp.float32),
        scratch_shapes=[
            pltpu.VMEM((SEQ, DIM), jnp.float32),
            pltpu.SemaphoreType.DMA,
        ],
    )(E_object_index)
